# baseline (speedup 1.0000x reference)
import functools

import jax
import jax.numpy as jnp
from jax import lax
from jax.experimental import pallas as pl
from jax.experimental.pallas import tpu as pltpu
from jax.experimental.pallas import tpu_sc as plsc

NC, NS, L = 2, 16, 16
NW = NC * NS

BATCH, FIELDS, D = 4096, 26, 32
R = BATCH * FIELDS
RPW = R // NW
CHUNK = 128
NCHUNK = RPW // CHUNK
GROUPS = RPW // L


def _rsqrt(v):
    i = lax.bitcast_convert_type(v, jnp.int32)
    y = lax.bitcast_convert_type(jnp.int32(0x5F3759DF) - (i >> 1), jnp.float32)
    for _ in range(3):
        y = y * (1.5 - 0.5 * v * y * y)
    return y


_mesh = plsc.VectorSubcoreMesh(core_axis_name="c", subcore_axis_name="s")


@functools.partial(
    pl.kernel,
    out_type=jax.ShapeDtypeStruct((R, D), jnp.float32),
    mesh=_mesh,
    compiler_params=pltpu.CompilerParams(use_tc_tiling_on_sc=False),
    scratch_types=[
        pltpu.VMEM((NCHUNK, CHUNK), jnp.int32),
        pltpu.VMEM((RPW, D), jnp.float32),
        pltpu.VMEM((D,), jnp.float32),
        pltpu.VMEM((D,), jnp.float32),
        pltpu.SemaphoreType.DMA,
    ],
)
def _embed_ln(x_hbm, table_hbm, gamma_hbm, beta_hbm, out_hbm,
              idx_v, rows_v, gamma_v, beta_v, sem):
    wid = lax.axis_index("s") * NC + lax.axis_index("c")

    pltpu.sync_copy(x_hbm.at[wid], idx_v)
    pltpu.sync_copy(gamma_hbm, gamma_v)
    pltpu.sync_copy(beta_hbm, beta_v)

    copies = []
    for j in range(NCHUNK):
        copies.append(
            pltpu.async_copy(
                table_hbm.at[idx_v.at[j]],
                rows_v.at[pl.ds(j * CHUNK, CHUNK)],
                sem,
            )
        )
    for cp in copies:
        cp.wait()

    g_lo = gamma_v[pl.ds(0, L)]
    g_hi = gamma_v[pl.ds(L, L)]
    b_lo = beta_v[pl.ds(0, L)]
    b_hi = beta_v[pl.ds(L, L)]

    U = 4

    lane = lax.iota(jnp.int32, L)
    perms = [lane ^ (1 << k) for k in range(4)]
    _dnums = lax.GatherDimensionNumbers(
        offset_dims=(), collapsed_slice_dims=(0,), start_index_map=(0,))

    def lane_perm(v, p):
        return lax.gather(v, p[:, None], _dnums, (1,),
                          mode=lax.GatherScatterMode.PROMISE_IN_BOUNDS)

    def allreduce_sum(v):
        for p in perms:
            v = v + lane_perm(v, p)
        return v

    def ln_one_row(r):
        a = rows_v[r, pl.ds(0, L)]
        b = rows_v[r, pl.ds(L, L)]
        total = allreduce_sum(a + b)
        total2 = allreduce_sum(a * a + b * b)
        mean = total * (1.0 / D)
        var = total2 * (1.0 / D) - mean * mean
        rstd = _rsqrt(var + 1e-5)
        rows_v[r, pl.ds(0, L)] = (a - mean) * rstd * g_lo + b_lo
        rows_v[r, pl.ds(L, L)] = (b - mean) * rstd * g_hi + b_hi

    def apply_body(i, _):
        for u in range(U):
            ln_one_row(i * U + u)
        return 0

    lax.fori_loop(0, RPW // U, apply_body, 0)

    pltpu.sync_copy(rows_v, out_hbm.at[pl.ds(wid * RPW, RPW)])


def kernel(x, table, gamma, beta):
    x2d = x.reshape(NW, NCHUNK, CHUNK)
    out = _embed_ln(x2d, table, gamma, beta)
    return out.reshape(BATCH, FIELDS * D)
